# Initial kernel scaffold; baseline (speedup 1.0000x reference)
import jax
import jax.numpy as jnp
from jax.experimental import pallas as pl


def kernel(x, W_enc, b_enc, W1, b1, W2, b2, W3, b3, ln_w, ln_b):
    raise NotImplementedError("write your pallas kernel here")


# speedup vs baseline: 4.7662x; 4.7662x over previous
import functools

import jax
import jax.numpy as jnp
from jax.experimental import pallas as pl
from jax.experimental.pallas import tpu as pltpu

_MT = 1024


def _gelu_exact(x):
    return 0.5 * x * (1.0 + jax.lax.erf(x * 0.7071067811865476))


def _fused_kernel(x_ref, wenc_ref, benc_ref, w1_ref, b1_ref, w2_ref, b2_ref,
                  w3_ref, b3_ref, lnw_ref, lnb_ref, out_ref):
    f32 = jnp.float32
    xb = x_ref[...].astype(jnp.bfloat16)
    logits = jnp.dot(xb, wenc_ref[...], preferred_element_type=f32)
    logits = logits + benc_ref[...]
    bits = jnp.sign(logits).astype(jnp.bfloat16)
    h = jnp.dot(bits, w1_ref[...], preferred_element_type=f32) + b1_ref[...]
    h = _gelu_exact(h).astype(jnp.bfloat16)
    h = jnp.dot(h, w2_ref[...], preferred_element_type=f32) + b2_ref[...]
    h = _gelu_exact(h).astype(jnp.bfloat16)
    h = jnp.dot(h, w3_ref[...], preferred_element_type=f32) + b3_ref[...]
    mean = jnp.mean(h, axis=-1, keepdims=True)
    cent = h - mean
    var = jnp.mean(cent * cent, axis=-1, keepdims=True)
    out_ref[...] = cent * jax.lax.rsqrt(var + 1e-5) * lnw_ref[...] + lnb_ref[...]


@functools.partial(jax.jit, static_argnames=())
def kernel(x, W_enc, b_enc, W1, b1, W2, b2, W3, b3, ln_w, ln_b):
    B, T, D = x.shape
    K = W_enc.shape[0]
    H = W1.shape[0]
    M = B * T
    xf = x.reshape(M, D)
    bf16 = jnp.bfloat16
    wencT = W_enc.T.astype(bf16)
    w1T = W1.T.astype(bf16)
    w2T = W2.T.astype(bf16)
    w3T = W3.T.astype(bf16)
    benc = b_enc.reshape(1, K)
    b1r = b1.reshape(1, H)
    b2r = b2.reshape(1, H)
    b3r = b3.reshape(1, D)
    lnw = ln_w.reshape(1, D)
    lnb = ln_b.reshape(1, D)

    grid = (M // _MT,)
    full = lambda shape: pl.BlockSpec(shape, lambda i: (0, 0))
    out = pl.pallas_call(
        _fused_kernel,
        grid=grid,
        in_specs=[
            pl.BlockSpec((_MT, D), lambda i: (i, 0)),
            full((D, K)), full((1, K)),
            full((K, H)), full((1, H)),
            full((H, H)), full((1, H)),
            full((H, D)), full((1, D)),
            full((1, D)), full((1, D)),
        ],
        out_specs=pl.BlockSpec((_MT, D), lambda i: (i, 0)),
        out_shape=jax.ShapeDtypeStruct((M, D), jnp.float32),
        compiler_params=pltpu.CompilerParams(
            dimension_semantics=("arbitrary",),
        ),
    )(xf, wencT, benc, w1T, b1r, w2T, b2r, w3T, b3r, lnw, lnb)
    return out.reshape(B, T, D)
